# baseline (speedup 1.0000x reference)
import functools

import jax
import jax.numpy as jnp
from jax import lax
from jax.experimental import pallas as pl
from jax.experimental.pallas import tpu as pltpu
from jax.experimental.pallas import tpu_sc as plsc

D = 32
B = 4096
H = 200

NC, NS = 2, 16
NW = NC * NS
BPW = B // NW
G = 2
ROWS_G = G * H
NGROUPS = BPW // G
NBUF = 4

_mesh = plsc.VectorSubcoreMesh(core_axis_name="c", subcore_axis_name="s")


@functools.partial(
    pl.kernel,
    out_type=jax.ShapeDtypeStruct((B, D), jnp.float32),
    mesh=_mesh,
    scratch_types=[
        pltpu.VMEM((BPW * H,), jnp.int32),
        *[pltpu.VMEM((ROWS_G, D), jnp.float32) for _ in range(NBUF)],
        pltpu.VMEM((BPW, D), jnp.float32),
        *[pltpu.SemaphoreType.DMA for _ in range(NBUF)],
    ],
    compiler_params=pltpu.CompilerParams(use_tc_tiling_on_sc=False),
)
def _cbow_sc(x_hbm, w_hbm, out_hbm, idx_v, *rest):
    bufs = rest[:NBUF]
    out_v = rest[NBUF]
    sems = rest[NBUF + 1:]
    wid = lax.axis_index("s") * NC + lax.axis_index("c")
    base = wid * BPW
    pltpu.sync_copy(x_hbm.at[pl.ds(base * H, BPW * H)], idx_v)

    def gather(g, b):
        return pltpu.async_copy(
            w_hbm.at[idx_v.at[pl.ds(g * ROWS_G, ROWS_G)]], bufs[b], sems[b])

    copies = [gather(b, b) for b in range(NBUF)]
    for g in range(NGROUPS):
        cur = g % NBUF
        copies[cur].wait()
        buf = bufs[cur]
        for i in range(G):
            row0 = i * H

            def h_body(h, carry, buf=buf, row0=row0):
                a0, a1 = carry
                a0 = a0 + buf[row0 + h, pl.ds(0, 16)]
                a1 = a1 + buf[row0 + h, pl.ds(16, 16)]
                return a0, a1

            zero = jnp.zeros((16,), jnp.float32)
            a0, a1 = lax.fori_loop(0, H, h_body, (zero, zero), unroll=8)
            out_v[g * G + i, pl.ds(0, 16)] = a0
            out_v[g * G + i, pl.ds(16, 16)] = a1
        if g + NBUF < NGROUPS:
            copies[cur] = gather(g + NBUF, cur)

    pltpu.sync_copy(out_v, out_hbm.at[pl.ds(base, BPW)])


def kernel(x, W):
    flat_x = x.reshape(-1).astype(jnp.int32)
    return _cbow_sc(flat_x, W)
